# baseline (speedup 1.0000x reference)
import functools

import jax
import jax.numpy as jnp
from jax import lax
from jax.experimental import pallas as pl
from jax.experimental.pallas import tpu as pltpu
from jax.experimental.pallas import tpu_sc as plsc
from jax._src.pallas import mpmd as _mpmd

D = 1024
V = 65536
B = 8192
BM = 256
KC = 2048
NC, NS = 2, 16
NW = NC * NS
BPW = B // NW
CH = 32
NCHW = BPW // CH

_MESH = plsc.VectorSubcoreMesh(
    core_axis_name="c", subcore_axis_name="s", num_cores=NC, num_subcores=NS
)

_SC_SCRATCH = [
    pltpu.VMEM((NCHW, CH), jnp.int32),
    pltpu.VMEM((CH, D), jnp.float32),
    pltpu.VMEM((CH, D), jnp.float32),
    pltpu.VMEM((CH, D), jnp.float32),
    pltpu.SemaphoreType.DMA,
    pltpu.SemaphoreType.DMA,
    pltpu.SemaphoreType.DMA,
]


def _worker_id():
    return lax.axis_index("s") * NC + lax.axis_index("c")


def _gather_body(mem_h, idx_h, out_h, idx_v, buf0, buf1, buf2, sem0, sem1, sem2):
    wid = _worker_id()
    pltpu.sync_copy(idx_h.at[pl.ds(wid * NCHW, NCHW)], idx_v)
    bufs, sems = (buf0, buf1, buf2), (sem0, sem1, sem2)
    descs = [None, None, None]
    descs[0] = pltpu.async_copy(mem_h.at[idx_v.at[0]], bufs[0], sems[0])
    descs[1] = pltpu.async_copy(mem_h.at[idx_v.at[1]], bufs[1], sems[1])
    for ci in range(NCHW):
        if ci + 2 < NCHW:
            nb = (ci + 2) % 3
            descs[nb] = pltpu.async_copy(mem_h.at[idx_v.at[ci + 2]], bufs[nb], sems[nb])
        descs[ci % 3].wait()
        pltpu.sync_copy(bufs[ci % 3], out_h.at[pl.ds(wid * BPW + ci * CH, CH)])


_gather = pl.kernel(
    _gather_body,
    out_type=jax.ShapeDtypeStruct((B, D), jnp.float32),
    mesh=_MESH,
    scratch_types=_SC_SCRATCH,
    name="hfa_sc_gather",
)


def _scatter_body(mem_h, idx_h, wv_h, out_h, idx_v, buf0, buf1, buf2, sem0, sem1, sem2):
    del mem_h, buf2, sem2
    wid = _worker_id()
    pltpu.sync_copy(idx_h.at[pl.ds(wid * NCHW, NCHW)], idx_v)
    bufs, sems = (buf0, buf1), (sem0, sem1)
    descs = [None, None]
    descs[0] = pltpu.async_copy(wv_h.at[pl.ds(wid * BPW, CH)], bufs[0], sems[0])
    for ci in range(NCHW):
        if ci + 1 < NCHW:
            nb = (ci + 1) % 2
            descs[nb] = pltpu.async_copy(
                wv_h.at[pl.ds(wid * BPW + (ci + 1) * CH, CH)], bufs[nb], sems[nb]
            )
        descs[ci % 2].wait()
        pltpu.sync_copy(bufs[ci % 2], out_h.at[idx_v.at[ci]])


_scatter = _mpmd._mpmd_map(
    [(_MESH, _scatter_body)],
    out_types=jax.ShapeDtypeStruct((V, D), jnp.float32),
    input_output_aliases={0: 0},
    scratch_types=_SC_SCRATCH,
    name="hfa_sc_scatter",
)


def _copy_body(mem_ref, w1_ref, wf1_ref, wf2_ref,
               out_ref, w1o_ref, wf1o_ref, wf2o_ref):
    out_ref[...] = mem_ref[...]
    w1o_ref[...] = w1_ref[...].astype(jnp.bfloat16)
    wf1o_ref[...] = wf1_ref[...].astype(jnp.bfloat16)
    wf2o_ref[...] = wf2_ref[...].astype(jnp.bfloat16)


_copy = pl.pallas_call(
    _copy_body,
    grid=(12,),
    in_specs=[
        pl.BlockSpec((2048, D), lambda m: (m + 20, 0)),
        pl.BlockSpec((256, D), lambda m: (jnp.minimum(m, 7), 0)),
        pl.BlockSpec((384, 2 * D), lambda m: (jnp.minimum(m, 7), 0)),
        pl.BlockSpec((256, D), lambda m: (jnp.minimum(m, 7), 0)),
    ],
    out_specs=[
        pl.BlockSpec((2048, D), lambda m: (m + 20, 0)),
        pl.BlockSpec((256, D), lambda m: (jnp.minimum(m, 7), 0)),
        pl.BlockSpec((384, 2 * D), lambda m: (jnp.minimum(m, 7), 0)),
        pl.BlockSpec((256, D), lambda m: (jnp.minimum(m, 7), 0)),
    ],
    out_shape=[
        jax.ShapeDtypeStruct((V, D), jnp.float32),
        jax.ShapeDtypeStruct((2 * D, D), jnp.bfloat16),
        jax.ShapeDtypeStruct((3 * D, 2 * D), jnp.bfloat16),
        jax.ShapeDtypeStruct((2 * D, D), jnp.bfloat16),
    ],
    name="hfa_tc_basecopy",
)


def _fused_body(val_ref, prev_ref, w1v_ref, w1p_ref, b1_ref, w2t_ref,
                b2_ref, fp_ref, fv_ref, fg_ref, bf1_ref, wf2_ref, bf2_ref,
                idxc_ref, idxr_ref, memblk_ref, base_in_ref,
                wv_ref, base_ref, d16_ref):
    del base_in_ref
    base_ref[...] = memblk_ref[...]
    m = pl.program_id(0)
    nb1 = B // BM

    @pl.when(m < nb1)
    def _mlp_phase():
        xv = val_ref[...]
        xp = prev_ref[...]
        xv16 = xv.astype(jnp.bfloat16)
        xp16 = xp.astype(jnp.bfloat16)
        h = jnp.maximum(
            jnp.dot(xv16, w1v_ref[...], preferred_element_type=jnp.float32)
            + jnp.dot(xp16, w1p_ref[...], preferred_element_type=jnp.float32)
            + b1_ref[...],
            0.0,
        )
        glogit = jnp.sum(h * w2t_ref[...], axis=1, keepdims=True) + b2_ref[0, 0]
        gate = jax.nn.sigmoid(glogit)
        pg16 = (xp * gate).astype(jnp.bfloat16)
        u = jnp.maximum(
            jnp.dot(xp16, fp_ref[...], preferred_element_type=jnp.float32)
            + jnp.dot(xv16, fv_ref[...], preferred_element_type=jnp.float32)
            + jnp.dot(pg16, fg_ref[...], preferred_element_type=jnp.float32)
            + bf1_ref[...],
            0.0,
        )
        interp = jnp.tanh(
            jnp.dot(u.astype(jnp.bfloat16), wf2_ref[...],
                    preferred_element_type=jnp.float32)
            + bf2_ref[...]
        )
        row = (m % nb1) * BM
        d16_ref[pl.ds(row, BM), :] = (gate * (interp - xp)).astype(jnp.bfloat16)

    @pl.when(m >= nb1)
    def _combine_phase():
        me = idxc_ref[:, 0:1].astype(jnp.int16)
        acc = jnp.zeros((BM, D), jnp.float32)
        for c in range(B // KC):
            ks = idxr_ref[0, :, pl.ds(c * KC, KC)].astype(jnp.int16)
            a = (me == ks).astype(jnp.bfloat16)
            acc = acc + jnp.dot(a, d16_ref[pl.ds(c * KC, KC), :],
                                preferred_element_type=jnp.float32)
        wv_ref[...] = prev_ref[...] + acc


def _const2(i, j):
    return lambda m: (i, j)


def _phase_blk(m):
    nb1 = B // BM
    return (jnp.where(m < nb1, m, m - nb1), 0)


_fused = pl.pallas_call(
    _fused_body,
    grid=(2 * (B // BM),),
    in_specs=[
        pl.BlockSpec((BM, D), _phase_blk),
        pl.BlockSpec((BM, D), _phase_blk),
        pl.BlockSpec((D, D), _const2(0, 0)),
        pl.BlockSpec((D, D), _const2(1, 0)),
        pl.BlockSpec((1, D), _const2(0, 0)),
        pl.BlockSpec((1, D), _const2(0, 0)),
        pl.BlockSpec((1, 128), _const2(0, 0)),
        pl.BlockSpec((D, 2 * D), _const2(0, 0)),
        pl.BlockSpec((D, 2 * D), _const2(1, 0)),
        pl.BlockSpec((D, 2 * D), _const2(2, 0)),
        pl.BlockSpec((1, 2 * D), _const2(0, 0)),
        pl.BlockSpec((2 * D, D), _const2(0, 0)),
        pl.BlockSpec((1, D), _const2(0, 0)),
        pl.BlockSpec((BM, 128), _phase_blk),
        pl.BlockSpec((1, 1, B), lambda m: (0, 0, 0)),
        pl.BlockSpec((640, D), lambda m: (m, 0)),
        pl.BlockSpec(memory_space=pltpu.HBM),
    ],
    out_specs=[
        pl.BlockSpec((BM, D), lambda m: (jnp.maximum(m - B // BM, 0), 0)),
        pl.BlockSpec((640, D), lambda m: (m, 0)),
    ],
    out_shape=[
        jax.ShapeDtypeStruct((B, D), jnp.float32),
        jax.ShapeDtypeStruct((V, D), jnp.float32),
    ],
    scratch_shapes=[pltpu.VMEM((B, D), jnp.bfloat16)],
    input_output_aliases={16: 1},
    name="hfa_tc_fused",
)


def kernel(mem, idx, val, W_sd1, b_sd1, W_sd2, b_sd2, W_fi1, b_fi1, W_fi2, b_fi2):
    idx32 = idx.astype(jnp.int32)
    idx2 = idx32.reshape(B // CH, CH)

    prev = _gather(mem, idx2)

    idx_mcol = jnp.broadcast_to(idx32[:, None], (B, 128))
    idx_row3 = idx32.reshape(1, 1, B)
    base0, w1_16, wf1_16, wf2_16 = _copy(mem, W_sd1, W_fi1, W_fi2)
    wv, base1 = _fused(
        val, prev,
        w1_16, w1_16,
        b_sd1.reshape(1, D),
        W_sd2.reshape(1, D),
        jnp.broadcast_to(b_sd2.reshape(1, 1), (1, 128)),
        wf1_16, wf1_16, wf1_16,
        b_fi1.reshape(1, 2 * D),
        wf2_16,
        b_fi2.reshape(1, D),
        idx_mcol, idx_row3, mem, base0,
    )

    return _scatter(base1, idx2, wv)


# speedup vs baseline: 1.0344x; 1.0344x over previous
import functools

import jax
import jax.numpy as jnp
from jax import lax
from jax.experimental import pallas as pl
from jax.experimental.pallas import tpu as pltpu
from jax.experimental.pallas import tpu_sc as plsc
from jax._src.pallas import mpmd as _mpmd

D = 1024
V = 65536
B = 8192
BM = 256
KC = 2048
NC, NS = 2, 16
NW = NC * NS
BPW = B // NW
CH = 32
NCHW = BPW // CH

_MESH = plsc.VectorSubcoreMesh(
    core_axis_name="c", subcore_axis_name="s", num_cores=NC, num_subcores=NS
)

_SC_SCRATCH = [
    pltpu.VMEM((NCHW, CH), jnp.int32),
    pltpu.VMEM((CH, D), jnp.float32),
    pltpu.VMEM((CH, D), jnp.float32),
    pltpu.VMEM((CH, D), jnp.float32),
    pltpu.SemaphoreType.DMA,
    pltpu.SemaphoreType.DMA,
    pltpu.SemaphoreType.DMA,
]


def _worker_id():
    return lax.axis_index("s") * NC + lax.axis_index("c")


def _gather_body(mem_h, idx_h, out_h, idx_v, buf0, buf1, buf2, sem0, sem1, sem2):
    wid = _worker_id()
    pltpu.sync_copy(idx_h.at[pl.ds(wid * NCHW, NCHW)], idx_v)
    bufs, sems = (buf0, buf1, buf2), (sem0, sem1, sem2)
    descs = [None, None, None]
    descs[0] = pltpu.async_copy(mem_h.at[idx_v.at[0]], bufs[0], sems[0])
    descs[1] = pltpu.async_copy(mem_h.at[idx_v.at[1]], bufs[1], sems[1])
    for ci in range(NCHW):
        if ci + 2 < NCHW:
            nb = (ci + 2) % 3
            descs[nb] = pltpu.async_copy(mem_h.at[idx_v.at[ci + 2]], bufs[nb], sems[nb])
        descs[ci % 3].wait()
        pltpu.sync_copy(bufs[ci % 3], out_h.at[pl.ds(wid * BPW + ci * CH, CH)])


_gather = pl.kernel(
    _gather_body,
    out_type=jax.ShapeDtypeStruct((B, D), jnp.float32),
    mesh=_MESH,
    scratch_types=_SC_SCRATCH,
    name="hfa_sc_gather",
)


def _scatter_body(mem_h, idx_h, wv_h, out_h, idx_v, buf0, buf1, buf2, sem0, sem1, sem2):
    del mem_h, buf2, sem2
    wid = _worker_id()
    pltpu.sync_copy(idx_h.at[pl.ds(wid * NCHW, NCHW)], idx_v)
    bufs, sems = (buf0, buf1), (sem0, sem1)
    descs = [None, None]
    descs[0] = pltpu.async_copy(wv_h.at[pl.ds(wid * BPW, CH)], bufs[0], sems[0])
    for ci in range(NCHW):
        if ci + 1 < NCHW:
            nb = (ci + 1) % 2
            descs[nb] = pltpu.async_copy(
                wv_h.at[pl.ds(wid * BPW + (ci + 1) * CH, CH)], bufs[nb], sems[nb]
            )
        descs[ci % 2].wait()
        pltpu.sync_copy(bufs[ci % 2], out_h.at[idx_v.at[ci]])


_scatter = _mpmd._mpmd_map(
    [(_MESH, _scatter_body)],
    out_types=jax.ShapeDtypeStruct((V, D), jnp.float32),
    input_output_aliases={0: 0},
    scratch_types=_SC_SCRATCH,
    name="hfa_sc_scatter",
)


def _copy_body(mem_ref, w1_ref, wf1_ref, wf2_ref,
               out_ref, w1o_ref, wf1o_ref, wf2o_ref):
    out_ref[...] = mem_ref[...]
    w1o_ref[...] = w1_ref[...].astype(jnp.bfloat16)
    wf1o_ref[...] = wf1_ref[...].astype(jnp.bfloat16)
    wf2o_ref[...] = wf2_ref[...].astype(jnp.bfloat16)


_copy = pl.pallas_call(
    _copy_body,
    grid=(8,),
    in_specs=[
        pl.BlockSpec((2048, D), lambda m: (m + 24, 0)),
        pl.BlockSpec((256, D), lambda m: (jnp.minimum(m, 7), 0)),
        pl.BlockSpec((384, 2 * D), lambda m: (jnp.minimum(m, 7), 0)),
        pl.BlockSpec((256, D), lambda m: (jnp.minimum(m, 7), 0)),
    ],
    out_specs=[
        pl.BlockSpec((2048, D), lambda m: (m + 24, 0)),
        pl.BlockSpec((256, D), lambda m: (jnp.minimum(m, 7), 0)),
        pl.BlockSpec((384, 2 * D), lambda m: (jnp.minimum(m, 7), 0)),
        pl.BlockSpec((256, D), lambda m: (jnp.minimum(m, 7), 0)),
    ],
    out_shape=[
        jax.ShapeDtypeStruct((V, D), jnp.float32),
        jax.ShapeDtypeStruct((2 * D, D), jnp.bfloat16),
        jax.ShapeDtypeStruct((3 * D, 2 * D), jnp.bfloat16),
        jax.ShapeDtypeStruct((2 * D, D), jnp.bfloat16),
    ],
    name="hfa_tc_basecopy",
)


def _fused_body(val_ref, prev_ref, w1v_ref, w1p_ref, b1_ref, w2t_ref,
                b2_ref, fp_ref, fv_ref, fg_ref, bf1_ref, wf2_ref, bf2_ref,
                idxc_ref, idxr_ref, memblk_ref, base_in_ref,
                wv_ref, base_ref, d16_ref):
    del base_in_ref
    base_ref[...] = memblk_ref[...]
    m = pl.program_id(0)
    nb1 = B // BM

    @pl.when(m < nb1)
    def _mlp_phase():
        xv = val_ref[...]
        xp = prev_ref[...]
        xv16 = xv.astype(jnp.bfloat16)
        xp16 = xp.astype(jnp.bfloat16)
        h = jnp.maximum(
            jnp.dot(xv16, w1v_ref[...], preferred_element_type=jnp.float32)
            + jnp.dot(xp16, w1p_ref[...], preferred_element_type=jnp.float32)
            + b1_ref[...],
            0.0,
        )
        glogit = jnp.sum(h * w2t_ref[...], axis=1, keepdims=True) + b2_ref[0, 0]
        gate = jax.nn.sigmoid(glogit)
        pg16 = (xp * gate).astype(jnp.bfloat16)
        u = jnp.maximum(
            jnp.dot(xp16, fp_ref[...], preferred_element_type=jnp.float32)
            + jnp.dot(xv16, fv_ref[...], preferred_element_type=jnp.float32)
            + jnp.dot(pg16, fg_ref[...], preferred_element_type=jnp.float32)
            + bf1_ref[...],
            0.0,
        )
        interp = jnp.tanh(
            jnp.dot(u.astype(jnp.bfloat16), wf2_ref[...],
                    preferred_element_type=jnp.float32)
            + bf2_ref[...]
        )
        row = (m % nb1) * BM
        d16_ref[pl.ds(row, BM), :] = (gate * (interp - xp)).astype(jnp.bfloat16)

    @pl.when(m >= nb1)
    def _combine_phase():
        me = idxc_ref[:, 0:1].astype(jnp.int16)
        acc = jnp.zeros((BM, D), jnp.float32)
        for c in range(B // KC):
            ks = idxr_ref[0, :, pl.ds(c * KC, KC)].astype(jnp.int16)
            a = (me == ks).astype(jnp.bfloat16)
            acc = acc + jnp.dot(a, d16_ref[pl.ds(c * KC, KC), :],
                                preferred_element_type=jnp.float32)
        wv_ref[...] = prev_ref[...] + acc


def _const2(i, j):
    return lambda m: (i, j)


def _phase_blk(m):
    nb1 = B // BM
    return (jnp.where(m < nb1, m, m - nb1), 0)


_fused = pl.pallas_call(
    _fused_body,
    grid=(2 * (B // BM),),
    in_specs=[
        pl.BlockSpec((BM, D), _phase_blk),
        pl.BlockSpec((BM, D), _phase_blk),
        pl.BlockSpec((D, D), _const2(0, 0)),
        pl.BlockSpec((D, D), _const2(1, 0)),
        pl.BlockSpec((1, D), _const2(0, 0)),
        pl.BlockSpec((1, D), _const2(0, 0)),
        pl.BlockSpec((1, 128), _const2(0, 0)),
        pl.BlockSpec((D, 2 * D), _const2(0, 0)),
        pl.BlockSpec((D, 2 * D), _const2(1, 0)),
        pl.BlockSpec((D, 2 * D), _const2(2, 0)),
        pl.BlockSpec((1, 2 * D), _const2(0, 0)),
        pl.BlockSpec((2 * D, D), _const2(0, 0)),
        pl.BlockSpec((1, D), _const2(0, 0)),
        pl.BlockSpec((BM, 128), _phase_blk),
        pl.BlockSpec((1, 1, B), lambda m: (0, 0, 0)),
        pl.BlockSpec((768, D), lambda m: (m, 0)),
        pl.BlockSpec(memory_space=pltpu.HBM),
    ],
    out_specs=[
        pl.BlockSpec((BM, D), lambda m: (jnp.maximum(m - B // BM, 0), 0)),
        pl.BlockSpec((768, D), lambda m: (m, 0)),
    ],
    out_shape=[
        jax.ShapeDtypeStruct((B, D), jnp.float32),
        jax.ShapeDtypeStruct((V, D), jnp.float32),
    ],
    scratch_shapes=[pltpu.VMEM((B, D), jnp.bfloat16)],
    input_output_aliases={16: 1},
    name="hfa_tc_fused",
)


def kernel(mem, idx, val, W_sd1, b_sd1, W_sd2, b_sd2, W_fi1, b_fi1, W_fi2, b_fi2):
    idx32 = idx.astype(jnp.int32)
    idx2 = idx32.reshape(B // CH, CH)

    prev = _gather(mem, idx2)

    idx_mcol = jnp.broadcast_to(idx32[:, None], (B, 128))
    idx_row3 = idx32.reshape(1, 1, B)
    base0, w1_16, wf1_16, wf2_16 = _copy(mem, W_sd1, W_fi1, W_fi2)
    wv, base1 = _fused(
        val, prev,
        w1_16, w1_16,
        b_sd1.reshape(1, D),
        W_sd2.reshape(1, D),
        jnp.broadcast_to(b_sd2.reshape(1, 1), (1, 128)),
        wf1_16, wf1_16, wf1_16,
        b_fi1.reshape(1, 2 * D),
        wf2_16,
        b_fi2.reshape(1, D),
        idx_mcol, idx_row3, mem, base0,
    )

    return _scatter(base1, idx2, wv)


# speedup vs baseline: 1.0744x; 1.0387x over previous
import functools

import jax
import jax.numpy as jnp
from jax import lax
from jax.experimental import pallas as pl
from jax.experimental.pallas import tpu as pltpu
from jax.experimental.pallas import tpu_sc as plsc
from jax._src.pallas import mpmd as _mpmd

D = 1024
V = 65536
B = 8192
BM = 256
KC = 2048
NC, NS = 2, 16
NW = NC * NS
BPW = B // NW
CH = 32
NCHW = BPW // CH

_MESH = plsc.VectorSubcoreMesh(
    core_axis_name="c", subcore_axis_name="s", num_cores=NC, num_subcores=NS
)

_SC_SCRATCH = [
    pltpu.VMEM((NCHW, CH), jnp.int32),
    pltpu.VMEM((CH, D), jnp.float32),
    pltpu.VMEM((CH, D), jnp.float32),
    pltpu.VMEM((CH, D), jnp.float32),
    pltpu.SemaphoreType.DMA,
    pltpu.SemaphoreType.DMA,
    pltpu.SemaphoreType.DMA,
]


def _worker_id():
    return lax.axis_index("s") * NC + lax.axis_index("c")


def _gather_body(mem_h, idx_h, out_h, idx_v, buf0, buf1, buf2, sem0, sem1, sem2):
    wid = _worker_id()
    pltpu.sync_copy(idx_h.at[pl.ds(wid * NCHW, NCHW)], idx_v)
    bufs, sems = (buf0, buf1, buf2), (sem0, sem1, sem2)
    descs = [None, None, None]
    descs[0] = pltpu.async_copy(mem_h.at[idx_v.at[0]], bufs[0], sems[0])
    descs[1] = pltpu.async_copy(mem_h.at[idx_v.at[1]], bufs[1], sems[1])
    for ci in range(NCHW):
        if ci + 2 < NCHW:
            nb = (ci + 2) % 3
            descs[nb] = pltpu.async_copy(mem_h.at[idx_v.at[ci + 2]], bufs[nb], sems[nb])
        descs[ci % 3].wait()
        pltpu.sync_copy(bufs[ci % 3], out_h.at[pl.ds(wid * BPW + ci * CH, CH)])


_gather = pl.kernel(
    _gather_body,
    out_type=jax.ShapeDtypeStruct((B, D), jnp.float32),
    mesh=_MESH,
    scratch_types=_SC_SCRATCH,
    name="hfa_sc_gather",
)


def _scatter_body(mem_h, idx_h, wv_h, out_h, idx_v, buf0, buf1, buf2, sem0, sem1, sem2):
    del mem_h, buf2, sem2
    wid = _worker_id()
    pltpu.sync_copy(idx_h.at[pl.ds(wid * NCHW, NCHW)], idx_v)
    bufs, sems = (buf0, buf1), (sem0, sem1)
    descs = [None, None]
    descs[0] = pltpu.async_copy(wv_h.at[pl.ds(wid * BPW, CH)], bufs[0], sems[0])
    for ci in range(NCHW):
        if ci + 1 < NCHW:
            nb = (ci + 1) % 2
            descs[nb] = pltpu.async_copy(
                wv_h.at[pl.ds(wid * BPW + (ci + 1) * CH, CH)], bufs[nb], sems[nb]
            )
        descs[ci % 2].wait()
        pltpu.sync_copy(bufs[ci % 2], out_h.at[idx_v.at[ci]])


_scatter = _mpmd._mpmd_map(
    [(_MESH, _scatter_body)],
    out_types=jax.ShapeDtypeStruct((V, D), jnp.float32),
    input_output_aliases={0: 0},
    scratch_types=_SC_SCRATCH,
    name="hfa_sc_scatter",
)


def _copy_body(mem_ref, w1_ref, wf1_ref, wf2_ref,
               out_ref, w1o_ref, wf1o_ref, wf2o_ref):
    out_ref[...] = mem_ref[...]
    w1o_ref[...] = w1_ref[...].astype(jnp.bfloat16)
    wf1o_ref[...] = wf1_ref[...].astype(jnp.bfloat16)
    wf2o_ref[...] = wf2_ref[...].astype(jnp.bfloat16)


_copy = pl.pallas_call(
    _copy_body,
    grid=(8,),
    in_specs=[
        pl.BlockSpec((1024, D), lambda m: (m + 56, 0)),
        pl.BlockSpec((256, D), lambda m: (jnp.minimum(m, 7), 0)),
        pl.BlockSpec((384, 2 * D), lambda m: (jnp.minimum(m, 7), 0)),
        pl.BlockSpec((256, D), lambda m: (jnp.minimum(m, 7), 0)),
    ],
    out_specs=[
        pl.BlockSpec((1024, D), lambda m: (m + 56, 0)),
        pl.BlockSpec((256, D), lambda m: (jnp.minimum(m, 7), 0)),
        pl.BlockSpec((384, 2 * D), lambda m: (jnp.minimum(m, 7), 0)),
        pl.BlockSpec((256, D), lambda m: (jnp.minimum(m, 7), 0)),
    ],
    out_shape=[
        jax.ShapeDtypeStruct((V, D), jnp.float32),
        jax.ShapeDtypeStruct((2 * D, D), jnp.bfloat16),
        jax.ShapeDtypeStruct((3 * D, 2 * D), jnp.bfloat16),
        jax.ShapeDtypeStruct((2 * D, D), jnp.bfloat16),
    ],
    name="hfa_tc_basecopy",
)


def _fused_body(val_ref, prev_ref, w1v_ref, w1p_ref, b1_ref, w2t_ref,
                b2_ref, fp_ref, fv_ref, fg_ref, bf1_ref, wf2_ref, bf2_ref,
                idxc_ref, idxr_ref, memblk_ref, base_in_ref,
                wv_ref, base_ref, d16_ref):
    del base_in_ref
    base_ref[...] = memblk_ref[...]
    m = pl.program_id(0)
    nb1 = B // BM

    @pl.when(m < nb1)
    def _mlp_phase():
        xv = val_ref[...]
        xp = prev_ref[...]
        xv16 = xv.astype(jnp.bfloat16)
        xp16 = xp.astype(jnp.bfloat16)
        h = jnp.maximum(
            jnp.dot(xv16, w1v_ref[...], preferred_element_type=jnp.float32)
            + jnp.dot(xp16, w1p_ref[...], preferred_element_type=jnp.float32)
            + b1_ref[...],
            0.0,
        )
        glogit = jnp.sum(h * w2t_ref[...], axis=1, keepdims=True) + b2_ref[0, 0]
        gate = jax.nn.sigmoid(glogit)
        pg16 = (xp * gate).astype(jnp.bfloat16)
        u = jnp.maximum(
            jnp.dot(xp16, fp_ref[...], preferred_element_type=jnp.float32)
            + jnp.dot(xv16, fv_ref[...], preferred_element_type=jnp.float32)
            + jnp.dot(pg16, fg_ref[...], preferred_element_type=jnp.float32)
            + bf1_ref[...],
            0.0,
        )
        interp = jnp.tanh(
            jnp.dot(u.astype(jnp.bfloat16), wf2_ref[...],
                    preferred_element_type=jnp.float32)
            + bf2_ref[...]
        )
        row = (m % nb1) * BM
        d16_ref[pl.ds(row, BM), :] = (gate * (interp - xp)).astype(jnp.bfloat16)

    @pl.when(m >= nb1)
    def _combine_phase():
        me = idxc_ref[:, 0:1].astype(jnp.int16)
        acc = jnp.zeros((BM, D), jnp.float32)
        for c in range(B // KC):
            ks = idxr_ref[0, :, pl.ds(c * KC, KC)].astype(jnp.int16)
            a = (me == ks).astype(jnp.bfloat16)
            acc = acc + jnp.dot(a, d16_ref[pl.ds(c * KC, KC), :],
                                preferred_element_type=jnp.float32)
        wv_ref[...] = prev_ref[...] + acc


def _const2(i, j):
    return lambda m: (i, j)


def _phase_blk(m):
    nb1 = B // BM
    return (jnp.where(m < nb1, m, m - nb1), 0)


_fused = pl.pallas_call(
    _fused_body,
    grid=(2 * (B // BM),),
    in_specs=[
        pl.BlockSpec((BM, D), _phase_blk),
        pl.BlockSpec((BM, D), _phase_blk),
        pl.BlockSpec((D, D), _const2(0, 0)),
        pl.BlockSpec((D, D), _const2(1, 0)),
        pl.BlockSpec((1, D), _const2(0, 0)),
        pl.BlockSpec((1, D), _const2(0, 0)),
        pl.BlockSpec((1, 128), _const2(0, 0)),
        pl.BlockSpec((D, 2 * D), _const2(0, 0)),
        pl.BlockSpec((D, 2 * D), _const2(1, 0)),
        pl.BlockSpec((D, 2 * D), _const2(2, 0)),
        pl.BlockSpec((1, 2 * D), _const2(0, 0)),
        pl.BlockSpec((2 * D, D), _const2(0, 0)),
        pl.BlockSpec((1, D), _const2(0, 0)),
        pl.BlockSpec((BM, 128), _phase_blk),
        pl.BlockSpec((1, 1, B), lambda m: (0, 0, 0)),
        pl.BlockSpec((896, D), lambda m: (m, 0)),
        pl.BlockSpec(memory_space=pltpu.HBM),
    ],
    out_specs=[
        pl.BlockSpec((BM, D), lambda m: (jnp.maximum(m - B // BM, 0), 0)),
        pl.BlockSpec((896, D), lambda m: (m, 0)),
    ],
    out_shape=[
        jax.ShapeDtypeStruct((B, D), jnp.float32),
        jax.ShapeDtypeStruct((V, D), jnp.float32),
    ],
    scratch_shapes=[pltpu.VMEM((B, D), jnp.bfloat16)],
    input_output_aliases={16: 1},
    name="hfa_tc_fused",
)


def kernel(mem, idx, val, W_sd1, b_sd1, W_sd2, b_sd2, W_fi1, b_fi1, W_fi2, b_fi2):
    idx32 = idx.astype(jnp.int32)
    idx2 = idx32.reshape(B // CH, CH)

    prev = _gather(mem, idx2)

    idx_mcol = jnp.broadcast_to(idx32[:, None], (B, 128))
    idx_row3 = idx32.reshape(1, 1, B)
    base0, w1_16, wf1_16, wf2_16 = _copy(mem, W_sd1, W_fi1, W_fi2)
    wv, base1 = _fused(
        val, prev,
        w1_16, w1_16,
        b_sd1.reshape(1, D),
        W_sd2.reshape(1, D),
        jnp.broadcast_to(b_sd2.reshape(1, 1), (1, 128)),
        wf1_16, wf1_16, wf1_16,
        b_fi1.reshape(1, 2 * D),
        wf2_16,
        b_fi2.reshape(1, D),
        idx_mcol, idx_row3, mem, base0,
    )

    return _scatter(base1, idx2, wv)
